# baseline (speedup 1.0000x reference)
import numpy as np
import jax
import jax.numpy as jnp
from jax import lax
from jax.experimental import pallas as pl

ACTN = 18
LANG = 32000
N = 1024


def _np_rotl(x, d):
    x = np.uint32(x)
    return np.uint32((np.uint32(x << np.uint32(d)) | np.uint32(x >> np.uint32(32 - d))))


def _np_threefry_pair(k0, k1, c0, c1):
    x0, x1 = np.uint32(c0), np.uint32(c1)
    ks = [np.uint32(k0), np.uint32(k1), np.uint32(np.uint32(k0) ^ np.uint32(k1) ^ np.uint32(0x1BD11BDA))]
    rot = [[13, 15, 26, 6], [17, 29, 16, 24]]
    x0 = np.uint32(x0 + ks[0])
    x1 = np.uint32(x1 + ks[1])
    for i in range(5):
        for r in rot[i % 2]:
            x0 = np.uint32(x0 + x1)
            x1 = _np_rotl(x1, r)
            x1 = np.uint32(x1 ^ x0)
        x0 = np.uint32(x0 + ks[(i + 1) % 3])
        x1 = np.uint32(x1 + ks[(i + 2) % 3] + np.uint32(i + 1))
    return int(x0), int(x1)


def _np_split(key):
    return _np_threefry_pair(key[0], key[1], 0, 0), _np_threefry_pair(key[0], key[1], 0, 1)


_ROOT = (0, 42)
_K1, _K2 = _np_split(_ROOT)
_K1A, _K1B = _np_split(_K1)
_K2A, _K2B = _np_split(_K2)


def _i32(v):
    v = int(v) & 0xFFFFFFFF
    return v - (1 << 32) if v >= (1 << 31) else v


def _rotl(x, d):
    return (x << d) | lax.shift_right_logical(x, 32 - d)


def _tf_bits(key, cnt):
    k0, k1 = _i32(key[0]), _i32(key[1])
    k2 = _i32((int(key[0]) ^ int(key[1]) ^ 0x1BD11BDA) & 0xFFFFFFFF)
    ks = (k0, k1, k2)
    rot = ((13, 15, 26, 6), (17, 29, 16, 24))
    x0 = jnp.full(cnt.shape, k0, jnp.int32)
    x1 = cnt + jnp.int32(k1)
    for i in range(5):
        for r in rot[i % 2]:
            x0 = x0 + x1
            x1 = _rotl(x1, r)
            x1 = x1 ^ x0
        x0 = x0 + jnp.int32(ks[(i + 1) % 3])
        x1 = x1 + jnp.int32(_i32((ks[(i + 2) % 3] + (i + 1)) & 0xFFFFFFFF))
    return x0 ^ x1


def _mod_const(x, s, bmax):
    for b in range(bmax, -1, -1):
        c = s << b
        x = jnp.where(x >= c, x - c, x)
    return x


def _mod16(x, s, bmax):
    return _mod_const(x, s, bmax)


def _randint_span(hbits, lbits, s, mult, bmax16, bmaxc, bmaxo):
    m16 = (1 << 16) % s

    def mod32(w):
        hi = lax.shift_right_logical(w, 16)
        lo = w & jnp.int32(0xFFFF)
        t = _mod16(hi, s, bmax16) * jnp.int32(m16) + _mod16(lo, s, bmax16)
        return _mod_const(t, s, bmaxc)

    off = mod32(hbits) * jnp.int32(mult) + mod32(lbits)
    return _mod_const(off, s, bmaxo)


def _sample_actn(cnt):
    h = _tf_bits(_K1A, cnt)
    l = _tf_bits(_K1B, cnt)
    return _randint_span(h, l, 18, 4, 11, 4, 2)


def _sample_lang(cnt):
    h = _tf_bits(_K2A, cnt)
    l = _tf_bits(_K2B, cnt)
    return _randint_span(h, l, 32000, 23296, 1, 10, 14)


_LANG_BLK = 6400


def _cnt_iota():
    return (lax.broadcasted_iota(jnp.int32, (8, 128), 0) * 128
            + lax.broadcasted_iota(jnp.int32, (8, 128), 1))


def _rng_kernel(ra_ref, rl_ref):
    cnt = _cnt_iota()
    ra_ref[...] = _sample_actn(cnt)
    rl_ref[...] = _sample_lang(cnt)


def _onehot_kernel(ra_ref, rl_ref, actn_ref, lang_ref):
    i = pl.program_id(0)
    cols = lax.broadcasted_iota(jnp.int32, (N, _LANG_BLK), 1) + i * _LANG_BLK
    lang_ref[...] = (cols == rl_ref[...]).astype(jnp.float32)

    @pl.when(i == 0)
    def _():
        acols = lax.broadcasted_iota(jnp.int32, (N, ACTN), 1)
        actn_ref[...] = (acols == ra_ref[...]).astype(jnp.float32)


def kernel(x):
    del x
    ra, rl = pl.pallas_call(
        _rng_kernel,
        out_shape=[
            jax.ShapeDtypeStruct((8, 128), jnp.int32),
            jax.ShapeDtypeStruct((8, 128), jnp.int32),
        ],
    )()
    ra2 = ra.reshape(N, 1)
    rl2 = rl.reshape(N, 1)
    actn, lang = pl.pallas_call(
        _onehot_kernel,
        grid=(LANG // _LANG_BLK,),
        in_specs=[
            pl.BlockSpec((N, 1), lambda i: (0, 0)),
            pl.BlockSpec((N, 1), lambda i: (0, 0)),
        ],
        out_specs=[
            pl.BlockSpec((N, ACTN), lambda i: (0, 0)),
            pl.BlockSpec((N, _LANG_BLK), lambda i: (0, i)),
        ],
        out_shape=[
            jax.ShapeDtypeStruct((N, ACTN), jnp.float32),
            jax.ShapeDtypeStruct((N, LANG), jnp.float32),
        ],
    )(ra2, rl2)
    return actn, (lang,)


# speedup vs baseline: 1.0531x; 1.0531x over previous
import numpy as np
import jax
import jax.numpy as jnp
from jax import lax
from jax.experimental import pallas as pl

ACTN = 18
LANG = 32000
N = 1024


def _np_rotl(x, d):
    x = np.uint32(x)
    return np.uint32((np.uint32(x << np.uint32(d)) | np.uint32(x >> np.uint32(32 - d))))


def _np_threefry_pair(k0, k1, c0, c1):
    x0, x1 = np.uint32(c0), np.uint32(c1)
    ks = [np.uint32(k0), np.uint32(k1), np.uint32(np.uint32(k0) ^ np.uint32(k1) ^ np.uint32(0x1BD11BDA))]
    rot = [[13, 15, 26, 6], [17, 29, 16, 24]]
    x0 = np.uint32(x0 + ks[0])
    x1 = np.uint32(x1 + ks[1])
    for i in range(5):
        for r in rot[i % 2]:
            x0 = np.uint32(x0 + x1)
            x1 = _np_rotl(x1, r)
            x1 = np.uint32(x1 ^ x0)
        x0 = np.uint32(x0 + ks[(i + 1) % 3])
        x1 = np.uint32(x1 + ks[(i + 2) % 3] + np.uint32(i + 1))
    return int(x0), int(x1)


def _np_split(key):
    return _np_threefry_pair(key[0], key[1], 0, 0), _np_threefry_pair(key[0], key[1], 0, 1)


_ROOT = (0, 42)
_K1, _K2 = _np_split(_ROOT)
_K1A, _K1B = _np_split(_K1)
_K2A, _K2B = _np_split(_K2)


def _i32(v):
    v = int(v) & 0xFFFFFFFF
    return v - (1 << 32) if v >= (1 << 31) else v


def _rotl(x, d):
    return (x << d) | lax.shift_right_logical(x, 32 - d)


def _tf_bits(key, cnt):
    k0, k1 = _i32(key[0]), _i32(key[1])
    k2 = _i32((int(key[0]) ^ int(key[1]) ^ 0x1BD11BDA) & 0xFFFFFFFF)
    ks = (k0, k1, k2)
    rot = ((13, 15, 26, 6), (17, 29, 16, 24))
    x0 = jnp.full(cnt.shape, k0, jnp.int32)
    x1 = cnt + jnp.int32(k1)
    for i in range(5):
        for r in rot[i % 2]:
            x0 = x0 + x1
            x1 = _rotl(x1, r)
            x1 = x1 ^ x0
        x0 = x0 + jnp.int32(ks[(i + 1) % 3])
        x1 = x1 + jnp.int32(_i32((ks[(i + 2) % 3] + (i + 1)) & 0xFFFFFFFF))
    return x0 ^ x1


def _mod_const(x, s, bmax):
    for b in range(bmax, -1, -1):
        c = s << b
        x = jnp.where(x >= c, x - c, x)
    return x


def _mod16(x, s, bmax):
    return _mod_const(x, s, bmax)


def _randint_span(hbits, lbits, s, mult, bmax16, bmaxc, bmaxo):
    m16 = (1 << 16) % s

    def mod32(w):
        hi = lax.shift_right_logical(w, 16)
        lo = w & jnp.int32(0xFFFF)
        t = _mod16(hi, s, bmax16) * jnp.int32(m16) + _mod16(lo, s, bmax16)
        return _mod_const(t, s, bmaxc)

    off = mod32(hbits) * jnp.int32(mult) + mod32(lbits)
    return _mod_const(off, s, bmaxo)


def _sample_actn(cnt):
    h = _tf_bits(_K1A, cnt)
    l = _tf_bits(_K1B, cnt)
    return _randint_span(h, l, 18, 4, 11, 4, 2)


def _sample_lang(cnt):
    h = _tf_bits(_K2A, cnt)
    l = _tf_bits(_K2B, cnt)
    return _randint_span(h, l, 32000, 23296, 1, 10, 14)


_LANG_BLK = 1280


def _cnt_iota():
    return (lax.broadcasted_iota(jnp.int32, (8, 128), 0) * 128
            + lax.broadcasted_iota(jnp.int32, (8, 128), 1))


def _rng_kernel(ra_ref, rl_ref):
    cnt = _cnt_iota()
    ra_ref[...] = _sample_actn(cnt)
    rl_ref[...] = _sample_lang(cnt)


def _onehot_kernel(ra_ref, rl_ref, actn_ref, lang_ref):
    i = pl.program_id(0)
    cols = lax.broadcasted_iota(jnp.int32, (N, _LANG_BLK), 1) + i * _LANG_BLK
    lang_ref[...] = (cols == rl_ref[...]).astype(jnp.float32)

    @pl.when(i == 0)
    def _():
        acols = lax.broadcasted_iota(jnp.int32, (N, ACTN), 1)
        actn_ref[...] = (acols == ra_ref[...]).astype(jnp.float32)


def kernel(x):
    del x
    ra, rl = pl.pallas_call(
        _rng_kernel,
        out_shape=[
            jax.ShapeDtypeStruct((8, 128), jnp.int32),
            jax.ShapeDtypeStruct((8, 128), jnp.int32),
        ],
    )()
    ra2 = ra.reshape(N, 1)
    rl2 = rl.reshape(N, 1)
    actn, lang = pl.pallas_call(
        _onehot_kernel,
        grid=(LANG // _LANG_BLK,),
        in_specs=[
            pl.BlockSpec((N, 1), lambda i: (0, 0)),
            pl.BlockSpec((N, 1), lambda i: (0, 0)),
        ],
        out_specs=[
            pl.BlockSpec((N, ACTN), lambda i: (0, 0)),
            pl.BlockSpec((N, _LANG_BLK), lambda i: (0, i)),
        ],
        out_shape=[
            jax.ShapeDtypeStruct((N, ACTN), jnp.float32),
            jax.ShapeDtypeStruct((N, LANG), jnp.float32),
        ],
    )(ra2, rl2)
    return actn, (lang,)
